# baseline (speedup 1.0000x reference)
import functools

import jax
import jax.numpy as jnp
from jax.experimental import pallas as pl
from jax.experimental.pallas import tpu as pltpu

K = 128
D = 64
N = 131072
BETA = 0.5
BN = 16384


def _vq_body(z_ref, cb_ref, zq_ref, idx_ref, loss_ref, acc_ref):
    i = pl.program_id(0)
    z = z_ref[...]
    cb = cb_ref[...]

    s1 = jnp.sum(z * z, axis=1, keepdims=True)
    s2 = jnp.sum(cb * cb, axis=1)
    m = jax.lax.dot_general(
        z, cb, (((1,), (1,)), ((), ())),
        preferred_element_type=jnp.float32)
    d = s1 + s2[None, :] - 2.0 * m

    dmin = jnp.min(d, axis=1, keepdims=True)
    kiota_f = jax.lax.broadcasted_iota(
        jnp.int32, (BN, K), 1).astype(jnp.float32)
    idx_f = jnp.min(jnp.where(d == dmin, kiota_f, float(K)),
                    axis=1, keepdims=True)

    onehot = (kiota_f == idx_f).astype(jnp.float32)

    kvec = jax.lax.broadcasted_iota(
        jnp.int32, (1, K), 1).astype(jnp.float32)
    idx_row = jax.lax.dot_general(
        kvec, onehot, (((1,), (1,)), ((), ())),
        preferred_element_type=jnp.float32)
    idx_ref[...] = idx_row.reshape(BN).astype(jnp.int32)
    cb_hi = cb.astype(jnp.bfloat16).astype(jnp.float32)
    cb_lo = cb - cb_hi
    zq = (jax.lax.dot_general(
              onehot, cb_hi, (((1,), (0,)), ((), ())),
              preferred_element_type=jnp.float32)
          + jax.lax.dot_general(
              onehot, cb_lo, (((1,), (0,)), ((), ())),
              preferred_element_type=jnp.float32))

    zq_ref[...] = zq

    part = jnp.sum(dmin)

    @pl.when(i == 0)
    def _init():
        acc_ref[0, 0] = 0.0

    acc_ref[0, 0] += part

    @pl.when(i == pl.num_programs(0) - 1)
    def _fin():
        mean = acc_ref[0, 0] * (1.0 / (N * D))
        loss_ref[0, 0] = BETA * mean + mean


@functools.partial(jax.jit, static_argnames=())
def kernel(z, codebook):
    grid = N // BN
    zq, idx, loss = pl.pallas_call(
        _vq_body,
        grid=(grid,),
        in_specs=[
            pl.BlockSpec((BN, D), lambda i: (i, 0)),
            pl.BlockSpec((K, D), lambda i: (0, 0)),
        ],
        out_specs=[
            pl.BlockSpec((BN, D), lambda i: (i, 0)),
            pl.BlockSpec((BN,), lambda i: (i,)),
            pl.BlockSpec(memory_space=pltpu.SMEM),
        ],
        out_shape=[
            jax.ShapeDtypeStruct((N, D), jnp.float32),
            jax.ShapeDtypeStruct((N,), jnp.int32),
            jax.ShapeDtypeStruct((1, 1), jnp.float32),
        ],
        scratch_shapes=[pltpu.SMEM((1, 1), jnp.float32)],
        compiler_params=pltpu.CompilerParams(
            dimension_semantics=("arbitrary",)),
    )(z, codebook)
    return (zq, idx, loss[0, 0])
